# baseline (device time: 277401 ns/iter reference)
import functools

import jax
import jax.numpy as jnp
from jax import lax
from jax.experimental import pallas as pl
from jax.experimental.pallas import tpu as pltpu

N_DEV = 4
F_TILE = 128
N_LAYERS = 3
N_SLOTS = 2 * N_LAYERS
T_RS = 6


def _all_peer_barrier(my):
    barrier_sem = pltpu.get_barrier_semaphore()
    for k in range(1, N_DEV):
        peer = lax.rem(my + k, N_DEV)
        pl.semaphore_signal(
            barrier_sem, inc=1,
            device_id=(peer,), device_id_type=pl.DeviceIdType.MESH,
        )
    pl.semaphore_wait(barrier_sem, N_DEV - 1)


def _exit_barrier(my):
    @functools.partial(pl.run_scoped, sem=pltpu.SemaphoreType.REGULAR)
    def _(sem):
        for k in range(1, N_DEV):
            peer = lax.rem(my + k, N_DEV)
            pl.semaphore_signal(
                sem, inc=1,
                device_id=(peer,), device_id_type=pl.DeviceIdType.MESH,
            )
        pl.semaphore_wait(sem, N_DEV - 1)


def kernel(x, Win0, Wout0, Win1, Wout1, Win2, Wout2):
    m_per, d = x.shape
    b = N_DEV * m_per
    half = b // 2
    chunk = half // N_DEV
    f = Win0.shape[1]
    n_t = f // F_TILE

    def body(x_ref, win0, wout0, win1, wout1, win2, wout2, out_ref,
             xbuf, pbuf, rs_buf,
             agx_send, agx_recv, rs_send, rs_recv, ag_send, ag_recv):
        my = lax.axis_index("i")
        l = pl.program_id(0)
        hf = pl.program_id(1)
        t = pl.program_id(2)
        s = 2 * l + hf

        def rs_send_rdma(slot, hslot, k):
            peer = lax.rem(my + k, N_DEV)
            return pltpu.make_async_remote_copy(
                src_ref=pbuf.at[hslot, pl.ds(peer * chunk, chunk), :],
                dst_ref=rs_buf.at[slot, 3 - k],
                send_sem=rs_send.at[slot, k - 1],
                recv_sem=rs_recv.at[slot, 3 - k],
                device_id=(peer,),
                device_id_type=pl.DeviceIdType.MESH,
            )

        def rs_recv_rdma(slot, j):
            return pltpu.make_async_remote_copy(
                src_ref=rs_buf.at[slot, j],
                dst_ref=rs_buf.at[slot, j],
                send_sem=rs_send.at[slot, 0],
                recv_sem=rs_recv.at[slot, j],
                device_id=(my,),
                device_id_type=pl.DeviceIdType.MESH,
            )

        def ag_send_rdma(target, slot, base, k):
            peer = lax.rem(my + k, N_DEV)
            sl = target.at[pl.ds(base + my * chunk, chunk), :]
            return pltpu.make_async_remote_copy(
                src_ref=sl, dst_ref=sl,
                send_sem=ag_send.at[slot, k - 1],
                recv_sem=ag_recv.at[slot, 3 - k],
                device_id=(peer,),
                device_id_type=pl.DeviceIdType.MESH,
            )

        def ag_recv_rdma(target, slot, base, j):
            peer = lax.rem(my + 3 - j, N_DEV)
            sl = target.at[pl.ds(base + peer * chunk, chunk), :]
            return pltpu.make_async_remote_copy(
                src_ref=sl, dst_ref=sl,
                send_sem=ag_send.at[slot, 0],
                recv_sem=ag_recv.at[slot, j],
                device_id=(peer,),
                device_id_type=pl.DeviceIdType.MESH,
            )

        @pl.when((s == 0) & (t == 0))
        def _():
            _all_peer_barrier(my)
            xbuf[pl.ds(my * m_per, m_per), :] = x_ref[...]
            sends = []
            for k in range(1, N_DEV):
                peer = lax.rem(my + k, N_DEV)
                rdma = pltpu.make_async_remote_copy(
                    src_ref=xbuf.at[pl.ds(my * m_per, m_per), :],
                    dst_ref=xbuf.at[pl.ds(my * m_per, m_per), :],
                    send_sem=agx_send.at[k - 1],
                    recv_sem=agx_recv.at[3 - k],
                    device_id=(peer,),
                    device_id_type=pl.DeviceIdType.MESH,
                )
                rdma.start()
                sends.append(rdma)
            for j in range(N_DEV - 1):
                peer = lax.rem(my + 3 - j, N_DEV)
                recv = pltpu.make_async_remote_copy(
                    src_ref=xbuf.at[pl.ds(peer * m_per, m_per), :],
                    dst_ref=xbuf.at[pl.ds(peer * m_per, m_per), :],
                    send_sem=agx_send.at[0],
                    recv_sem=agx_recv.at[j],
                    device_id=(peer,),
                    device_id_type=pl.DeviceIdType.MESH,
                )
                recv.wait_recv()
            for rdma in sends:
                rdma.wait_send()

        @pl.when((t == 0) & (s >= 2))
        def _():
            sp = s - 2
            base = hf * half
            for j in range(N_DEV - 1):
                ag_recv_rdma(xbuf, sp, base, j).wait_recv()
            for k in range(1, N_DEV):
                rs_send_rdma(sp, hf, k).wait_send()
                ag_send_rdma(xbuf, sp, base, k).wait_send()

        win = [win0, win1, win2]
        wout = [wout0, wout1, wout2]
        x_half = xbuf[pl.ds(hf * half, half), :]
        for li in range(N_LAYERS):
            @pl.when(l == li)
            def _(li=li):
                hh = jnp.dot(x_half, win[li][...],
                             preferred_element_type=jnp.float32)
                hh = jnp.maximum(hh, 0.0)
                p = jnp.dot(hh, wout[li][...],
                            preferred_element_type=jnp.float32)

                @pl.when(t == 0)
                def _():
                    pbuf[hf] = p

                @pl.when(t > 0)
                def _():
                    pbuf[hf] += p

        @pl.when((t == T_RS) & (s >= 1))
        def _():
            sp = s - 1
            hp = 1 - hf
            lp = sp // 2
            base = hp * half
            for j in range(N_DEV - 1):
                rs_recv_rdma(sp, j).wait_recv()
            red = (pbuf[hp, pl.ds(my * chunk, chunk), :]
                   + rs_buf[sp, 0] + rs_buf[sp, 1] + rs_buf[sp, 2])

            def finish(target):
                target[pl.ds(base + my * chunk, chunk), :] = red
                for k in range(1, N_DEV):
                    ag_send_rdma(target, sp, base, k).start()

            @pl.when(lp < N_LAYERS - 1)
            def _():
                finish(xbuf)

            @pl.when(lp == N_LAYERS - 1)
            def _():
                finish(out_ref)

        @pl.when(t == n_t - 1)
        def _():
            for k in range(1, N_DEV):
                rs_send_rdma(s, hf, k).start()

            @pl.when(s == N_SLOTS - 1)
            def _():
                for j in range(N_DEV - 1):
                    ag_recv_rdma(out_ref, N_SLOTS - 2, 0, j).wait_recv()
                for j in range(N_DEV - 1):
                    rs_recv_rdma(N_SLOTS - 1, j).wait_recv()
                red = (pbuf[1, pl.ds(my * chunk, chunk), :]
                       + rs_buf[N_SLOTS - 1, 0]
                       + rs_buf[N_SLOTS - 1, 1]
                       + rs_buf[N_SLOTS - 1, 2])
                out_ref[pl.ds(half + my * chunk, chunk), :] = red
                ag5 = []
                for k in range(1, N_DEV):
                    rdma = ag_send_rdma(out_ref, N_SLOTS - 1, half, k)
                    rdma.start()
                    ag5.append(rdma)
                for j in range(N_DEV - 1):
                    ag_recv_rdma(out_ref, N_SLOTS - 1, half, j).wait_recv()
                for k in range(1, N_DEV):
                    rs_send_rdma(N_SLOTS - 2, 0, k).wait_send()
                    ag_send_rdma(out_ref, N_SLOTS - 2, 0, k).wait_send()
                    rs_send_rdma(N_SLOTS - 1, 1, k).wait_send()
                for rdma in ag5:
                    rdma.wait_send()
                _exit_barrier(my)

    frozen = n_t - 1

    def win_map(li):
        def m(l, hf, t):
            if li == 0:
                return (0, jnp.where(l == 0, t, frozen))
            if li == 1:
                return (0, jnp.where(l < 1, 0, jnp.where(l == 1, t, frozen)))
            return (0, jnp.where(l < 2, 0, t))
        return m

    def wout_map(li):
        def m(l, hf, t):
            if li == 0:
                return (jnp.where(l == 0, t, frozen), 0)
            if li == 1:
                return (jnp.where(l < 1, 0, jnp.where(l == 1, t, frozen)), 0)
            return (jnp.where(l < 2, 0, t), 0)
        return m

    return pl.pallas_call(
        body,
        grid=(N_LAYERS, 2, n_t),
        out_shape=jax.ShapeDtypeStruct((b, d), jnp.float32),
        in_specs=[
            pl.BlockSpec((m_per, d), lambda l, hf, t: (0, 0)),
            pl.BlockSpec((d, F_TILE), win_map(0)),
            pl.BlockSpec((F_TILE, d), wout_map(0)),
            pl.BlockSpec((d, F_TILE), win_map(1)),
            pl.BlockSpec((F_TILE, d), wout_map(1)),
            pl.BlockSpec((d, F_TILE), win_map(2)),
            pl.BlockSpec((F_TILE, d), wout_map(2)),
        ],
        out_specs=pl.BlockSpec((b, d), lambda l, hf, t: (0, 0)),
        scratch_shapes=[
            pltpu.VMEM((b, d), jnp.float32),
            pltpu.VMEM((2, half, d), jnp.float32),
            pltpu.VMEM((N_SLOTS, N_DEV - 1, chunk, d), jnp.float32),
            pltpu.SemaphoreType.DMA((N_DEV - 1,)),
            pltpu.SemaphoreType.DMA((N_DEV - 1,)),
            pltpu.SemaphoreType.DMA((N_SLOTS, N_DEV - 1)),
            pltpu.SemaphoreType.DMA((N_SLOTS, N_DEV - 1)),
            pltpu.SemaphoreType.DMA((N_SLOTS, N_DEV - 1)),
            pltpu.SemaphoreType.DMA((N_SLOTS, N_DEV - 1)),
        ],
        compiler_params=pltpu.CompilerParams(
            collective_id=0, vmem_limit_bytes=48 * 1024 * 1024,
        ),
    )(x, Win0, Wout0, Win1, Wout1, Win2, Wout2)


# device time: 242224 ns/iter; 1.1452x vs baseline; 1.1452x over previous
import functools

import jax
import jax.numpy as jnp
from jax import lax
from jax.experimental import pallas as pl
from jax.experimental.pallas import tpu as pltpu

N_DEV = 4
F_TILE = 128
N_LAYERS = 3
S_SPLIT = 16


def _all_peer_barrier(my):
    barrier_sem = pltpu.get_barrier_semaphore()
    for k in range(1, N_DEV):
        peer = lax.rem(my + k, N_DEV)
        pl.semaphore_signal(
            barrier_sem, inc=1,
            device_id=(peer,), device_id_type=pl.DeviceIdType.MESH,
        )
    pl.semaphore_wait(barrier_sem, N_DEV - 1)


def _exit_barrier(my):
    @functools.partial(pl.run_scoped, sem=pltpu.SemaphoreType.REGULAR)
    def _(sem):
        for k in range(1, N_DEV):
            peer = lax.rem(my + k, N_DEV)
            pl.semaphore_signal(
                sem, inc=1,
                device_id=(peer,), device_id_type=pl.DeviceIdType.MESH,
            )
        pl.semaphore_wait(sem, N_DEV - 1)


def kernel(x, Win0, Wout0, Win1, Wout1, Win2, Wout2):
    m_per, d = x.shape
    b = N_DEV * m_per
    f = Win0.shape[1]
    n_t = f // F_TILE
    bf16 = jnp.bfloat16

    def body(x_ref, win0, wout0, win1, wout1, win2, wout2, out_ref,
             xbuf, pbufa, pbufb, xstage, xag_buf,
             rsa_src, rsb_src, ag_src, rsa_buf, rsb_buf, ag_buf,
             agx_send, agx_recv, rsa_send, rsa_recv,
             rsb_send, rsb_recv, ag_send, ag_recv):
        my = lax.axis_index("i")
        l = pl.program_id(0)
        t = pl.program_id(1)

        def send_rdma(src, dst, ssem, rsem, k):
            peer = lax.rem(my + k, N_DEV)
            return pltpu.make_async_remote_copy(
                src_ref=src, dst_ref=dst,
                send_sem=ssem, recv_sem=rsem,
                device_id=(peer,), device_id_type=pl.DeviceIdType.MESH,
            )

        def recv_wait(buf_slice, rsem):
            pltpu.make_async_remote_copy(
                src_ref=buf_slice, dst_ref=buf_slice,
                send_sem=rsem, recv_sem=rsem,
                device_id=(my,), device_id_type=pl.DeviceIdType.MESH,
            ).wait_recv()

        @pl.when((l == 0) & (t == 0))
        def _():
            _all_peer_barrier(my)
            xbuf[pl.ds(my * m_per, m_per), :] = x_ref[...]
            xstage[...] = x_ref[...].astype(bf16)
            for k in range(1, N_DEV):
                send_rdma(xstage, xag_buf.at[3 - k],
                          agx_send.at[k - 1], agx_recv.at[3 - k], k).start()
            for j in range(N_DEV - 1):
                recv_wait(xag_buf.at[j], agx_recv.at[j])
                peer = lax.rem(my + j + 1, N_DEV)
                xbuf[pl.ds(peer * m_per, m_per), :] = (
                    xag_buf[j].astype(jnp.float32))

        @pl.when((t == 0) & (l >= 1))
        def _():
            lp = l - 1
            for j in range(N_DEV - 1):
                recv_wait(ag_buf.at[lp, j], ag_recv.at[lp, j])
                peer = lax.rem(my + j + 1, N_DEV)
                xbuf[pl.ds(peer * m_per, m_per), :] = (
                    ag_buf[lp, j].astype(jnp.float32))

        win = [win0, win1, win2]
        wout = [wout0, wout1, wout2]
        xv = xbuf[...]
        for li in range(N_LAYERS):
            @pl.when(l == li)
            def _(li=li):
                hh = jnp.dot(xv, win[li][...],
                             preferred_element_type=jnp.float32)
                hh = jnp.maximum(hh, 0.0)
                p = jnp.dot(hh, wout[li][...],
                            preferred_element_type=jnp.float32)

                @pl.when(t == 0)
                def _():
                    pbufa[...] = p

                @pl.when((t > 0) & (t < S_SPLIT))
                def _():
                    pbufa[...] += p

                @pl.when(t == S_SPLIT)
                def _():
                    pbufb[...] = p

                @pl.when(t > S_SPLIT)
                def _():
                    pbufb[...] += p

        @pl.when(t == S_SPLIT)
        def _():
            for k in range(1, N_DEV):
                peer = lax.rem(my + k, N_DEV)
                rsa_src[l, k - 1] = (
                    pbufa[pl.ds(peer * m_per, m_per), :].astype(bf16))
                send_rdma(rsa_src.at[l, k - 1], rsa_buf.at[l, 3 - k],
                          rsa_send.at[l, k - 1], rsa_recv.at[l, 3 - k],
                          k).start()

        @pl.when(t == n_t - 1)
        def _():
            for k in range(1, N_DEV):
                peer = lax.rem(my + k, N_DEV)
                rsb_src[l, k - 1] = (
                    pbufb[pl.ds(peer * m_per, m_per), :].astype(bf16))
                send_rdma(rsb_src.at[l, k - 1], rsb_buf.at[l, 3 - k],
                          rsb_send.at[l, k - 1], rsb_recv.at[l, 3 - k],
                          k).start()
            for j in range(N_DEV - 1):
                recv_wait(rsa_buf.at[l, j], rsa_recv.at[l, j])
            for j in range(N_DEV - 1):
                recv_wait(rsb_buf.at[l, j], rsb_recv.at[l, j])
            red = (pbufa[pl.ds(my * m_per, m_per), :]
                   + pbufb[pl.ds(my * m_per, m_per), :]
                   + (rsa_buf[l, 0].astype(jnp.float32)
                      + rsa_buf[l, 1].astype(jnp.float32)
                      + rsa_buf[l, 2].astype(jnp.float32))
                   + (rsb_buf[l, 0].astype(jnp.float32)
                      + rsb_buf[l, 1].astype(jnp.float32)
                      + rsb_buf[l, 2].astype(jnp.float32)))

            @pl.when(l < N_LAYERS - 1)
            def _():
                xbuf[pl.ds(my * m_per, m_per), :] = red

            @pl.when(l == N_LAYERS - 1)
            def _():
                out_ref[pl.ds(my * m_per, m_per), :] = red

            ag_src[l] = red.astype(bf16)
            for k in range(1, N_DEV):
                send_rdma(ag_src.at[l], ag_buf.at[l, 3 - k],
                          ag_send.at[l, k - 1], ag_recv.at[l, 3 - k],
                          k).start()

            @pl.when(l == N_LAYERS - 1)
            def _():
                lf = N_LAYERS - 1
                for j in range(N_DEV - 1):
                    recv_wait(ag_buf.at[lf, j], ag_recv.at[lf, j])
                    peer = lax.rem(my + j + 1, N_DEV)
                    out_ref[pl.ds(peer * m_per, m_per), :] = (
                        ag_buf[lf, j].astype(jnp.float32))
                for k in range(1, N_DEV):
                    send_rdma(xstage, xag_buf.at[3 - k],
                              agx_send.at[k - 1], agx_recv.at[3 - k],
                              k).wait_send()
                    for l2 in range(N_LAYERS):
                        send_rdma(rsa_src.at[l2, k - 1],
                                  rsa_buf.at[l2, 3 - k],
                                  rsa_send.at[l2, k - 1],
                                  rsa_recv.at[l2, 3 - k], k).wait_send()
                        send_rdma(rsb_src.at[l2, k - 1],
                                  rsb_buf.at[l2, 3 - k],
                                  rsb_send.at[l2, k - 1],
                                  rsb_recv.at[l2, 3 - k], k).wait_send()
                        send_rdma(ag_src.at[l2], ag_buf.at[l2, 3 - k],
                                  ag_send.at[l2, k - 1],
                                  ag_recv.at[l2, 3 - k], k).wait_send()
                _exit_barrier(my)

    frozen = n_t - 1

    def win_map(li):
        def m(l, t):
            if li == 0:
                return (0, jnp.where(l == 0, t, frozen))
            if li == 1:
                return (0, jnp.where(l < 1, 0, jnp.where(l == 1, t, frozen)))
            return (0, jnp.where(l < 2, 0, t))
        return m

    def wout_map(li):
        def m(l, t):
            if li == 0:
                return (jnp.where(l == 0, t, frozen), 0)
            if li == 1:
                return (jnp.where(l < 1, 0, jnp.where(l == 1, t, frozen)), 0)
            return (jnp.where(l < 2, 0, t), 0)
        return m

    return pl.pallas_call(
        body,
        grid=(N_LAYERS, n_t),
        out_shape=jax.ShapeDtypeStruct((b, d), jnp.float32),
        in_specs=[
            pl.BlockSpec((m_per, d), lambda l, t: (0, 0)),
            pl.BlockSpec((d, F_TILE), win_map(0)),
            pl.BlockSpec((F_TILE, d), wout_map(0)),
            pl.BlockSpec((d, F_TILE), win_map(1)),
            pl.BlockSpec((F_TILE, d), wout_map(1)),
            pl.BlockSpec((d, F_TILE), win_map(2)),
            pl.BlockSpec((F_TILE, d), wout_map(2)),
        ],
        out_specs=pl.BlockSpec((b, d), lambda l, t: (0, 0)),
        scratch_shapes=[
            pltpu.VMEM((b, d), jnp.float32),
            pltpu.VMEM((b, d), jnp.float32),
            pltpu.VMEM((b, d), jnp.float32),
            pltpu.VMEM((m_per, d), bf16),
            pltpu.VMEM((N_DEV - 1, m_per, d), bf16),
            pltpu.VMEM((N_LAYERS, N_DEV - 1, m_per, d), bf16),
            pltpu.VMEM((N_LAYERS, N_DEV - 1, m_per, d), bf16),
            pltpu.VMEM((N_LAYERS, m_per, d), bf16),
            pltpu.VMEM((N_LAYERS, N_DEV - 1, m_per, d), bf16),
            pltpu.VMEM((N_LAYERS, N_DEV - 1, m_per, d), bf16),
            pltpu.VMEM((N_LAYERS, N_DEV - 1, m_per, d), bf16),
            pltpu.SemaphoreType.DMA((N_DEV - 1,)),
            pltpu.SemaphoreType.DMA((N_DEV - 1,)),
            pltpu.SemaphoreType.DMA((N_LAYERS, N_DEV - 1)),
            pltpu.SemaphoreType.DMA((N_LAYERS, N_DEV - 1)),
            pltpu.SemaphoreType.DMA((N_LAYERS, N_DEV - 1)),
            pltpu.SemaphoreType.DMA((N_LAYERS, N_DEV - 1)),
            pltpu.SemaphoreType.DMA((N_LAYERS, N_DEV - 1)),
            pltpu.SemaphoreType.DMA((N_LAYERS, N_DEV - 1)),
        ],
        compiler_params=pltpu.CompilerParams(
            collective_id=0, vmem_limit_bytes=48 * 1024 * 1024,
        ),
    )(x, Win0, Wout0, Win1, Wout1, Win2, Wout2)
